# Initial kernel scaffold; baseline (speedup 1.0000x reference)
import jax
import jax.numpy as jnp
from jax.experimental import pallas as pl


def kernel(t, xs, h_initial, edges, emb_W, emb_b, out_W, out_b, eW1, eb1, eW2, eb2, nW1, nb1, nW2, nb2, cW1, cb1, cW2):
    raise NotImplementedError("write your pallas kernel here")


# speedup vs baseline: 20.7576x; 20.7576x over previous
import jax
import jax.numpy as jnp
from jax import lax
from jax.experimental import pallas as pl
from jax.experimental.pallas import tpu as pltpu

B, P, D, H, L = 1024, 22, 3, 64, 4
Pp = 24
Bb = 16
R = Bb * Pp * Pp


def _silu(v):
    return v * jax.nn.sigmoid(v)


def _egnn_block(t_ref, x_ref, hi_ref, embWh_ref, embWt_ref, embb_ref,
                eW1r_ref, eW1c_ref, eW1x_ref, eb1_ref, eW2_ref, eb2_ref,
                cW1_ref, cb1_ref, cW2_ref, nW1h_ref, nW1a_ref, nb1_ref,
                nW2_ref, nb2_ref, o_ref):
    f32 = jnp.float32
    x4 = x_ref[:]

    base = jnp.dot(hi_ref[:], embWh_ref[:], preferred_element_type=f32)
    base = base + embb_ref[:]
    tcol = t_ref[:]
    h3 = (lax.broadcast_in_dim(base, (Bb, Pp, H), (1, 2))
          + tcol[:, :, None] * embWt_ref[:].reshape(1, 1, H))
    hn = h3.reshape(Bb * Pp, H)

    i4 = lax.broadcasted_iota(jnp.int32, (1, Pp, Pp, 1), 1)
    j4 = lax.broadcasted_iota(jnp.int32, (1, Pp, Pp, 1), 2)
    emask4 = ((i4 < P) & (j4 < P) & (i4 != j4)).astype(f32)

    xi = lax.broadcast_in_dim(x4, (Bb, Pp, Pp, D), (0, 1, 3))
    xj = lax.broadcast_in_dim(x4, (Bb, Pp, Pp, D), (0, 2, 3))
    cd0 = xi - xj
    ea4 = jnp.sum(cd0 * cd0, axis=-1, keepdims=True)
    ea = ea4.reshape(R, 1)

    coord4 = x4
    for l in range(L):
        if l == 0:
            cd4, rad = cd0, ea
        else:
            ci = lax.broadcast_in_dim(coord4, (Bb, Pp, Pp, D), (0, 1, 3))
            cj = lax.broadcast_in_dim(coord4, (Bb, Pp, Pp, D), (0, 2, 3))
            cd4 = ci - cj
            rad = jnp.sum(cd4 * cd4, axis=-1, keepdims=True).reshape(R, 1)

        A = jnp.dot(hn, eW1r_ref[l], preferred_element_type=f32)
        C = jnp.dot(hn, eW1c_ref[l], preferred_element_type=f32)
        A3 = A.reshape(Bb, Pp, H)
        C3 = C.reshape(Bb, Pp, H)
        Ab = lax.broadcast_in_dim(A3, (Bb, Pp, Pp, H), (0, 1, 3)).reshape(R, H)
        Cb = lax.broadcast_in_dim(C3, (Bb, Pp, Pp, H), (0, 2, 3)).reshape(R, H)
        wx = eW1x_ref[l]
        z1 = Ab + Cb + rad * wx[0:1, :] + ea * wx[1:2, :] + eb1_ref[l]
        m1 = _silu(z1)
        m = _silu(jnp.dot(m1, eW2_ref[l], preferred_element_type=f32)
                  + eb2_ref[l])
        c1 = _silu(jnp.dot(m, cW1_ref[l], preferred_element_type=f32)
                   + cb1_ref[l])
        cm = jnp.sum(c1 * cW2_ref[l], axis=-1, keepdims=True)
        cm4 = cm.reshape(Bb, Pp, Pp, 1) * emask4
        coord4 = coord4 + jnp.sum(cd4 * cm4, axis=2)

        if l < L - 1:
            m4m = m.reshape(Bb, Pp, Pp, H) * emask4
            agg = jnp.sum(m4m, axis=2).reshape(Bb * Pp, H)
            zn = (jnp.dot(hn, nW1h_ref[l], preferred_element_type=f32)
                  + jnp.dot(agg, nW1a_ref[l], preferred_element_type=f32)
                  + nb1_ref[l])
            hn = hn + jnp.dot(_silu(zn), nW2_ref[l],
                              preferred_element_type=f32) + nb2_ref[l]

    vel4 = coord4 - x4
    nmask = (lax.broadcasted_iota(jnp.int32, (1, Pp, 1), 1) < P).astype(f32)
    mean = jnp.sum(vel4 * nmask, axis=1, keepdims=True) * (1.0 / P)
    o_ref[:] = vel4 - mean


def kernel(t, xs, h_initial, edges, emb_W, emb_b, out_W, out_b,
           eW1, eb1, eW2, eb2, nW1, nb1, nW2, nb2, cW1, cb1, cW2):
    del edges, out_W, out_b
    f32 = jnp.float32
    x = xs.reshape(B, P, D)
    xpad = jnp.pad(x, ((0, 0), (0, Pp - P), (0, 0)))
    hi_pad = jnp.pad(h_initial, ((0, Pp - P), (0, 0)))
    embWh = emb_W[:4]
    embWt = emb_W[4:5]
    embb = emb_b.reshape(1, H)
    eW1r = eW1[:, :H, :]
    eW1c = eW1[:, H:2 * H, :]
    eW1x = eW1[:, 2 * H:, :]
    eb1r = eb1.reshape(L, 1, H)
    eb2r = eb2.reshape(L, 1, H)
    nW1h = nW1[:, :H, :]
    nW1a = nW1[:, H:, :]
    nb1r = nb1.reshape(L, 1, H)
    nb2r = nb2.reshape(L, 1, H)
    cb1r = cb1.reshape(L, 1, H)
    cW2t = cW2.transpose(0, 2, 1)

    G = B // Bb
    full = lambda *shape: pl.BlockSpec(shape, lambda g: (0,) * len(shape))
    out = pl.pallas_call(
        _egnn_block,
        grid=(G,),
        in_specs=[
            pl.BlockSpec((Bb, 1), lambda g: (g, 0)),
            pl.BlockSpec((Bb, Pp, D), lambda g: (g, 0, 0)),
            full(Pp, 4),
            full(4, H), full(1, H), full(1, H),
            full(L, H, H), full(L, H, H), full(L, 2, H), full(L, 1, H),
            full(L, H, H), full(L, 1, H),
            full(L, H, H), full(L, 1, H), full(L, 1, H),
            full(L, H, H), full(L, H, H), full(L, 1, H),
            full(L, H, H), full(L, 1, H),
        ],
        out_specs=pl.BlockSpec((Bb, Pp, D), lambda g: (g, 0, 0)),
        out_shape=jax.ShapeDtypeStruct((B, Pp, D), f32),
        compiler_params=pltpu.CompilerParams(
            dimension_semantics=("parallel",)),
    )(t, xpad, hi_pad, embWh, embWt, embb, eW1r, eW1c, eW1x, eb1r,
      eW2, eb2r, cW1, cb1r, cW2t, nW1h, nW1a, nb1r, nW2, nb2r)
    return out[:, :P, :].reshape(B, P * D)


# speedup vs baseline: 22.6250x; 1.0900x over previous
import jax
import jax.numpy as jnp
from jax import lax
from jax.experimental import pallas as pl
from jax.experimental.pallas import tpu as pltpu

B, P, D, H, L = 1024, 22, 3, 64, 4
Pp = 24
Bb = 16
R = Bb * Pp * Pp


def _silu(v):
    return 0.5 * v * (jnp.tanh(0.5 * v) + 1.0)


def _egnn_block(t_ref, x_ref, hi_ref, embWh_ref, embWt_ref, embb_ref,
                eW1r_ref, eW1c_ref, Wr3_ref, We3_ref, eb1_ref, eW2_ref,
                eb2_ref, cW1_ref, cb1_ref, cW2_ref, nW1h_ref, nW1a_ref,
                nb1_ref, nW2_ref, nb2_ref, o_ref):
    f32 = jnp.float32
    x4 = x_ref[:]

    base = jnp.dot(hi_ref[:], embWh_ref[:], preferred_element_type=f32)
    base = base + embb_ref[:]
    tcol = t_ref[:]
    h3 = (lax.broadcast_in_dim(base, (Bb, Pp, H), (1, 2))
          + tcol[:, :, None] * embWt_ref[:].reshape(1, 1, H))
    hn = h3.reshape(Bb * Pp, H)

    nvalid = (lax.broadcasted_iota(jnp.int32, (1, Pp, 1), 1) < P)
    nmask3 = nvalid.astype(f32)
    nkill = (1.0 - nmask3) * (-1e30)

    xi = lax.broadcast_in_dim(x4, (Bb, Pp, Pp, D), (0, 1, 3))
    xj = lax.broadcast_in_dim(x4, (Bb, Pp, Pp, D), (0, 2, 3))
    cd0 = xi - xj
    cdf0 = cd0.reshape(R, D)
    cdsq0 = cdf0 * cdf0

    coord4 = x4
    for l in range(L):
        if l == 0:
            cd4, cdsq = cd0, cdsq0
        else:
            ci = lax.broadcast_in_dim(coord4, (Bb, Pp, Pp, D), (0, 1, 3))
            cj = lax.broadcast_in_dim(coord4, (Bb, Pp, Pp, D), (0, 2, 3))
            cd4 = ci - cj
            cdf = cd4.reshape(R, D)
            cdsq = cdf * cdf

        A = (jnp.dot(hn, eW1r_ref[l], preferred_element_type=f32)
             + eb1_ref[l])
        C = jnp.dot(hn, eW1c_ref[l], preferred_element_type=f32)
        A3 = A.reshape(Bb, Pp, H)
        C3 = C.reshape(Bb, Pp, H)
        me = _silu(eb2_ref[l])
        cstar = jnp.dot(_silu(jnp.dot(me, cW1_ref[l],
                                      preferred_element_type=f32)
                              + cb1_ref[l]),
                        cW2_ref[l], preferred_element_type=f32)
        m1d = _silu(A + C)
        md = _silu(jnp.dot(m1d, eW2_ref[l], preferred_element_type=f32)
                   + eb2_ref[l])
        A3k = A3 + nkill
        C3k = C3 + nkill
        Ab = lax.broadcast_in_dim(A3k, (Bb, Pp, Pp, H), (0, 1, 3)).reshape(R, H)
        Cb = lax.broadcast_in_dim(C3k, (Bb, Pp, Pp, H), (0, 2, 3)).reshape(R, H)
        z1 = (Ab + Cb
              + jnp.dot(cdsq, Wr3_ref[l], preferred_element_type=f32)
              + jnp.dot(cdsq0, We3_ref[l], preferred_element_type=f32))
        m1 = _silu(z1)
        m = _silu(jnp.dot(m1, eW2_ref[l], preferred_element_type=f32)
                  + eb2_ref[l])
        c1 = _silu(jnp.dot(m, cW1_ref[l], preferred_element_type=f32)
                   + cb1_ref[l])
        cm = jnp.dot(c1, cW2_ref[l], preferred_element_type=f32)
        cm4 = cm.reshape(Bb, Pp, Pp, 1)
        upd = jnp.sum(cd4 * cm4, axis=2)
        coord4 = (coord4 + upd
                  - 2.0 * cstar.reshape(1, 1, 1) * coord4) * nmask3

        if l < L - 1:
            agg = (jnp.sum(m.reshape(Bb, Pp, Pp, H), axis=2)
                   .reshape(Bb * Pp, H)) - md - 2.0 * me
            zn = (jnp.dot(hn, nW1h_ref[l], preferred_element_type=f32)
                  + jnp.dot(agg, nW1a_ref[l], preferred_element_type=f32)
                  + nb1_ref[l])
            hn = hn + jnp.dot(_silu(zn), nW2_ref[l],
                              preferred_element_type=f32) + nb2_ref[l]

    vel4 = coord4 - x4
    nmask = (lax.broadcasted_iota(jnp.int32, (1, Pp, 1), 1) < P).astype(f32)
    mean = jnp.sum(vel4 * nmask, axis=1, keepdims=True) * (1.0 / P)
    o_ref[:] = vel4 - mean


def kernel(t, xs, h_initial, edges, emb_W, emb_b, out_W, out_b,
           eW1, eb1, eW2, eb2, nW1, nb1, nW2, nb2, cW1, cb1, cW2):
    del edges, out_W, out_b
    f32 = jnp.float32
    x = xs.reshape(B, P, D)
    xpad = jnp.pad(x, ((0, 0), (0, Pp - P), (0, 0)))
    hi_pad = jnp.pad(h_initial, ((0, Pp - P), (0, 0)))
    embWh = emb_W[:4]
    embWt = emb_W[4:5]
    embb = emb_b.reshape(1, H)
    eW1r = eW1[:, :H, :]
    eW1c = eW1[:, H:2 * H, :]
    Wr3 = jnp.broadcast_to(eW1[:, 2 * H:2 * H + 1, :], (L, D, H))
    We3 = jnp.broadcast_to(eW1[:, 2 * H + 1:, :], (L, D, H))
    eb1r = eb1.reshape(L, 1, H)
    eb2r = eb2.reshape(L, 1, H)
    nW1h = nW1[:, :H, :]
    nW1a = nW1[:, H:, :]
    nb1r = nb1.reshape(L, 1, H)
    nb2r = nb2.reshape(L, 1, H)
    cb1r = cb1.reshape(L, 1, H)

    G = B // Bb
    full = lambda *shape: pl.BlockSpec(shape, lambda g: (0,) * len(shape))
    out = pl.pallas_call(
        _egnn_block,
        grid=(G,),
        in_specs=[
            pl.BlockSpec((Bb, 1), lambda g: (g, 0)),
            pl.BlockSpec((Bb, Pp, D), lambda g: (g, 0, 0)),
            full(Pp, 4),
            full(4, H), full(1, H), full(1, H),
            full(L, H, H), full(L, H, H), full(L, D, H), full(L, D, H),
            full(L, 1, H),
            full(L, H, H), full(L, 1, H),
            full(L, H, H), full(L, 1, H), full(L, H, 1),
            full(L, H, H), full(L, H, H), full(L, 1, H),
            full(L, H, H), full(L, 1, H),
        ],
        out_specs=pl.BlockSpec((Bb, Pp, D), lambda g: (g, 0, 0)),
        out_shape=jax.ShapeDtypeStruct((B, Pp, D), f32),
        compiler_params=pltpu.CompilerParams(
            dimension_semantics=("parallel",)),
    )(t, xpad, hi_pad, embWh, embWt, embb, eW1r, eW1c, Wr3, We3, eb1r,
      eW2, eb2r, cW1, cb1r, cW2, nW1h, nW1a, nb1r, nW2, nb2r)
    return out[:, :P, :].reshape(B, P * D)


# speedup vs baseline: 39.7526x; 1.7570x over previous
import numpy as np
import jax
import jax.numpy as jnp
from jax import lax
from jax.experimental import pallas as pl
from jax.experimental.pallas import tpu as pltpu

B, P, D, H, L = 1024, 22, 3, 64, 4
Pp = 24
Q = 8
W2 = 2 * H
NR = Q * Pp
R = Q * Pp * Pp

_PAT26 = np.kron(np.eye(2), np.ones((1, 3))).astype(np.float32)
_PAT2H = np.kron(np.eye(2), np.ones((1, H))).astype(np.float32)


def _sh(t):
    return t * (jnp.tanh(t) + 1.0)


def _egnn_block(t_ref, x_ref, hi_ref, embWh_ref, embWt_ref, embb_ref,
                p26_ref, p2h_ref,
                eW1r_ref, eW1c_ref, Wr3_ref, We3_ref, eb1_ref,
                eW2_ref, eb2_ref, cW1_ref, cb1_ref, cW2_ref,
                nW1h_ref, nW1a_ref, nb1_ref, nW2_ref, nb2_ref, o_ref):
    f32 = jnp.float32
    x2 = x_ref[:]

    base = jnp.dot(hi_ref[:], embWh_ref[:], preferred_element_type=f32)
    base = base + embb_ref[:]
    baseD = jnp.concatenate([base, base], axis=-1)
    t64 = jnp.dot(t_ref[:], p2h_ref[:], preferred_element_type=f32)
    wtD = jnp.concatenate([embWt_ref[:], embWt_ref[:]], axis=-1)
    h2 = (lax.broadcast_in_dim(baseD, (Q, Pp, W2), (1, 2))
          + lax.broadcast_in_dim(t64, (Q, Pp, W2), (0, 2))
          * wtD.reshape(1, 1, W2))
    hn = h2.reshape(NR, W2)

    nvalid = (lax.broadcasted_iota(jnp.int32, (1, Pp, 1), 1) < P)
    nmask3 = nvalid.astype(f32)
    nkill = (1.0 - nmask3) * (-1e30)

    def cdiff(c2):
        ci = lax.broadcast_in_dim(c2, (Q, Pp, Pp, 2 * D), (0, 1, 3))
        cj = lax.broadcast_in_dim(c2, (Q, Pp, Pp, 2 * D), (0, 2, 3))
        return ci - cj

    cd0 = cdiff(x2)
    cdsq0 = (cd0 * cd0).reshape(R, 2 * D)

    coord2 = x2
    for l in range(L):
        if l == 0:
            cd, cdsq = cd0, cdsq0
        else:
            cd = cdiff(coord2)
            cdsq = (cd * cd).reshape(R, 2 * D)

        A = (jnp.dot(hn, eW1r_ref[l], preferred_element_type=f32)
             + eb1_ref[l])
        C = jnp.dot(hn, eW1c_ref[l], preferred_element_type=f32)
        me = _sh(eb2_ref[l])
        cstar = jnp.dot(_sh(jnp.dot(me, cW1_ref[l],
                                    preferred_element_type=f32)
                            + cb1_ref[l]),
                        cW2_ref[l], preferred_element_type=f32)
        m1d = _sh(A + C)
        md = _sh(jnp.dot(m1d, eW2_ref[l], preferred_element_type=f32)
                 + eb2_ref[l])
        A3 = A.reshape(Q, Pp, W2)
        C3 = C.reshape(Q, Pp, W2) + nkill
        Ab = lax.broadcast_in_dim(A3, (Q, Pp, Pp, W2), (0, 1, 3)).reshape(R, W2)
        Cb = lax.broadcast_in_dim(C3, (Q, Pp, Pp, W2), (0, 2, 3)).reshape(R, W2)
        z1 = (Ab + Cb
              + jnp.dot(cdsq, Wr3_ref[l], preferred_element_type=f32)
              + jnp.dot(cdsq0, We3_ref[l], preferred_element_type=f32))
        m1 = _sh(z1)
        m = _sh(jnp.dot(m1, eW2_ref[l], preferred_element_type=f32)
                + eb2_ref[l])
        c1 = _sh(jnp.dot(m, cW1_ref[l], preferred_element_type=f32)
                 + cb1_ref[l])
        cmp = jnp.dot(c1, cW2_ref[l], preferred_element_type=f32)
        cm6 = jnp.dot(cmp, p26_ref[:], preferred_element_type=f32)
        upd = jnp.sum(cd * cm6.reshape(Q, Pp, Pp, 2 * D), axis=2)
        cs6 = jnp.dot(cstar, p26_ref[:],
                      preferred_element_type=f32).reshape(1, 1, 2 * D)
        coord2 = (coord2 + upd - 2.0 * cs6 * coord2) * nmask3

        if l < L - 1:
            agg = (jnp.sum(m.reshape(Q, Pp, Pp, W2), axis=2)
                   .reshape(NR, W2)) - md - 2.0 * me
            zn = (jnp.dot(hn, nW1h_ref[l], preferred_element_type=f32)
                  + jnp.dot(agg, nW1a_ref[l], preferred_element_type=f32)
                  + nb1_ref[l])
            hn = hn + jnp.dot(_sh(zn), nW2_ref[l],
                              preferred_element_type=f32) + nb2_ref[l]

    vel2 = coord2 - x2
    mean = jnp.sum(vel2 * nmask3, axis=1, keepdims=True) * (1.0 / P)
    o_ref[:] = vel2 - mean


def _blkdiag(Wl):
    z = jnp.zeros_like(Wl)
    top = jnp.concatenate([Wl, z], axis=-1)
    bot = jnp.concatenate([z, Wl], axis=-1)
    return jnp.concatenate([top, bot], axis=1)


def _dup(bl):
    return jnp.concatenate([bl, bl], axis=-1)


def kernel(t, xs, h_initial, edges, emb_W, emb_b, out_W, out_b,
           eW1, eb1, eW2, eb2, nW1, nb1, nW2, nb2, cW1, cb1, cW2):
    del edges, out_W, out_b
    f32 = jnp.float32
    half = B // 2
    x = xs.reshape(B, P, D)
    xpad = jnp.pad(x, ((0, 0), (0, Pp - P), (0, 0)))
    xpack = (xpad.reshape(half, 2, Pp, D).transpose(0, 2, 1, 3)
             .reshape(half, Pp, 2 * D))
    tpack = t.reshape(half, 2)
    hi_pad = jnp.pad(h_initial, ((0, Pp - P), (0, 0)))

    embWh = emb_W[:4]
    embWt = emb_W[4:5]
    embb = emb_b.reshape(1, H)
    eW1r = _blkdiag(0.5 * eW1[:, :H, :])
    eW1c = _blkdiag(0.5 * eW1[:, H:2 * H, :])
    Wr3 = _blkdiag(jnp.broadcast_to(0.5 * eW1[:, 2 * H:2 * H + 1, :],
                                    (L, D, H)))
    We3 = _blkdiag(jnp.broadcast_to(0.5 * eW1[:, 2 * H + 1:, :], (L, D, H)))
    eb1r = _dup(0.5 * eb1.reshape(L, 1, H))
    eW2d = _blkdiag(0.5 * eW2)
    eb2d = _dup(0.5 * eb2.reshape(L, 1, H))
    cW1d = _blkdiag(0.5 * cW1)
    cb1d = _dup(0.5 * cb1.reshape(L, 1, H))
    cW2d = _blkdiag(cW2)
    nW1h = _blkdiag(0.5 * nW1[:, :H, :])
    nW1a = _blkdiag(0.5 * nW1[:, H:, :])
    nb1r = _dup(0.5 * nb1.reshape(L, 1, H))
    nW2d = _blkdiag(nW2)
    nb2r = _dup(nb2.reshape(L, 1, H))
    p26 = jnp.asarray(_PAT26)
    p2h = jnp.asarray(_PAT2H)

    G = half // Q
    full = lambda *shape: pl.BlockSpec(shape, lambda g: (0,) * len(shape))
    out = pl.pallas_call(
        _egnn_block,
        grid=(G,),
        in_specs=[
            pl.BlockSpec((Q, 2), lambda g: (g, 0)),
            pl.BlockSpec((Q, Pp, 2 * D), lambda g: (g, 0, 0)),
            full(Pp, 4),
            full(4, H), full(1, H), full(1, H),
            full(2, 2 * D), full(2, W2),
            full(L, W2, W2), full(L, W2, W2),
            full(L, 2 * D, W2), full(L, 2 * D, W2), full(L, 1, W2),
            full(L, W2, W2), full(L, 1, W2),
            full(L, W2, W2), full(L, 1, W2), full(L, W2, 2),
            full(L, W2, W2), full(L, W2, W2), full(L, 1, W2),
            full(L, W2, W2), full(L, 1, W2),
        ],
        out_specs=pl.BlockSpec((Q, Pp, 2 * D), lambda g: (g, 0, 0)),
        out_shape=jax.ShapeDtypeStruct((half, Pp, 2 * D), f32),
        compiler_params=pltpu.CompilerParams(
            dimension_semantics=("parallel",)),
    )(tpack, xpack, hi_pad, embWh, embWt, embb, p26, p2h,
      eW1r, eW1c, Wr3, We3, eb1r, eW2d, eb2d, cW1d, cb1d, cW2d,
      nW1h, nW1a, nb1r, nW2d, nb2r)
    vel = (out.reshape(half, Pp, 2, D).transpose(0, 2, 1, 3)
           .reshape(B, Pp, D))
    return vel[:, :P, :].reshape(B, P * D)
